# baseline (speedup 1.0000x reference)
import functools

import jax
import jax.numpy as jnp
from jax import lax
from jax.experimental import pallas as pl
from jax.experimental.pallas import tpu as pltpu
from jax.experimental.pallas import tpu_sc as plsc

N_NODES = 10000
N_EDGES = 320000
DIM = 128

NC = 2
NS = 16
NT = NC * NS

NPAD = 10240
ZROW = NPAD - 1
NQ = 4
QR = NPAD // NQ
CH = 128
CAP = 10240
CAPCH = CAP // CH
E_TILE = N_EDGES // NT
HCH = 2000
ROWS_T = NPAD // NS
QROWS_T = QR // NS

_mesh = plsc.VectorSubcoreMesh(core_axis_name="c", subcore_axis_name="s")
_sc_params = pltpu.CompilerParams(needs_layout_passes=False)


@functools.partial(
    pl.kernel,
    out_type=jax.ShapeDtypeStruct((NC * NPAD,), jnp.float32),
    mesh=_mesh,
    compiler_params=_sc_params,
    scratch_types=[
        pltpu.VMEM((NPAD,), jnp.float32),
        pltpu.VMEM((HCH,), jnp.int32),
        pltpu.VMEM((NS, ROWS_T), jnp.float32),
        pltpu.VMEM((ROWS_T,), jnp.float32),
        pltpu.VMEM_SHARED((NS, NPAD), jnp.float32),
    ],
)
def _deg_fn(col_hbm, out_hbm, hist_v, colc_v, red_v, outb_v, stage_sh):
    c = lax.axis_index("c")
    s = lax.axis_index("s")
    gwid = c * NS + s
    zeros16 = jnp.zeros((16,), jnp.float32)
    ones16 = jnp.ones((16,), jnp.float32)

    def zbody(i, carry):
        hist_v[pl.ds(i * 16, 16)] = zeros16
        return carry

    lax.fori_loop(0, NPAD // 16, zbody, 0)

    def chunk_body(ci, carry):
        pltpu.sync_copy(col_hbm.at[pl.ds(gwid * E_TILE + ci * HCH, HCH)],
                        colc_v)

        def ibody(j, icarry):
            idx = colc_v[pl.ds(j * 16, 16)]
            plsc.addupdate_scatter(hist_v, [idx], ones16)
            return icarry

        lax.fori_loop(0, HCH // 16, ibody, 0)
        return carry

    lax.fori_loop(0, E_TILE // HCH, chunk_body, 0)

    pltpu.sync_copy(hist_v, stage_sh.at[s])
    plsc.subcore_barrier()

    pltpu.sync_copy(stage_sh.at[:, pl.ds(s * ROWS_T, ROWS_T)], red_v)

    def rbody(i, carry):
        acc = red_v[0, pl.ds(i * 16, 16)]
        for k in range(1, NS):
            acc = acc + red_v[k, pl.ds(i * 16, 16)]
        outb_v[pl.ds(i * 16, 16)] = acc
        return carry

    lax.fori_loop(0, ROWS_T // 16, rbody, 0)
    pltpu.sync_copy(outb_v, out_hbm.at[pl.ds(c * NPAD + s * ROWS_T, ROWS_T)])


@functools.partial(
    pl.kernel,
    out_type=[
        jax.ShapeDtypeStruct((NQ * NT * CAP,), jnp.int32),
        jax.ShapeDtypeStruct((NQ * NT * CAP,), jnp.int32),
        jax.ShapeDtypeStruct((NT, 16), jnp.int32),
    ],
    mesh=_mesh,
    compiler_params=_sc_params,
    scratch_types=[
        pltpu.VMEM((HCH,), jnp.int32),
        pltpu.VMEM((HCH,), jnp.int32),
        pltpu.VMEM((NQ, CAP), jnp.int32),
        pltpu.VMEM((NQ, CAP), jnp.int32),
        pltpu.VMEM((16,), jnp.int32),
    ],
)
def _part_fn(row_hbm, col_hbm, colq_hbm, rowq_hbm, cnts_hbm,
             cstage, rstage, qcol, qrow, cntv):
    c = lax.axis_index("c")
    s = lax.axis_index("s")
    gwid = c * NS + s
    colnull = jnp.full((16,), ZROW, jnp.int32)
    rownull = jnp.zeros((16,), jnp.int32)

    def chunk_body(ci, cnts):
        off = gwid * E_TILE + ci * HCH
        pltpu.sync_copy(col_hbm.at[pl.ds(off, HCH)], cstage)
        pltpu.sync_copy(row_hbm.at[pl.ds(off, HCH)], rstage)

        def ibody(j, icnts):
            cv = cstage[pl.ds(j * 16, 16)]
            rv = rstage[pl.ds(j * 16, 16)]
            out = []
            for q in range(NQ):
                lo = q * QR
                mq = (rv >= lo) & (rv < lo + QR)
                cq = icnts[q]
                plsc.store_compressed(qcol.at[q, pl.ds(cq, 16)], cv, mask=mq)
                plsc.store_compressed(qrow.at[q, pl.ds(cq, 16)], rv - lo, mask=mq)
                pc = plsc.all_reduce_population_count(mq)
                out.append(cq + jnp.max(pc))
            return tuple(out)

        return lax.fori_loop(0, HCH // 16, ibody, cnts)

    z = jnp.int32(0)
    cnts = lax.fori_loop(0, E_TILE // HCH, chunk_body, (z, z, z, z))

    for q in range(NQ):
        for k in range(CH // 16):
            qcol[q, pl.ds(cnts[q] + k * 16, 16)] = colnull
            qrow[q, pl.ds(cnts[q] + k * 16, 16)] = rownull

    for q in range(NQ):
        base = (q * NT + gwid) * CAP
        pltpu.sync_copy(qcol.at[q], colq_hbm.at[pl.ds(base, CAP)])
        pltpu.sync_copy(qrow.at[q], rowq_hbm.at[pl.ds(base, CAP)])

    lanes = lax.iota(jnp.int32, 16)
    cvec = jnp.where(lanes == 0, cnts[0],
                     jnp.where(lanes == 1, cnts[1],
                               jnp.where(lanes == 2, cnts[2],
                                         jnp.where(lanes == 3, cnts[3], 0))))
    cntv[pl.ds(0, 16)] = cvec
    pltpu.sync_copy(cntv, cnts_hbm.at[gwid])


@functools.partial(
    pl.kernel,
    out_type=jax.ShapeDtypeStruct((NPAD, DIM), jnp.float32),
    mesh=_mesh,
    compiler_params=_sc_params,
    scratch_types=[
        pltpu.VMEM((CH,), jnp.int32),
        pltpu.VMEM((CH,), jnp.int32),
        pltpu.VMEM((CH, DIM), jnp.float32),
        pltpu.VMEM((16,), jnp.int32),
        pltpu.VMEM_SHARED((NPAD, DIM), jnp.float32),
        pltpu.VMEM_SHARED((QR, DIM), jnp.float32),
    ],
)
def _spmm_fn(x_hbm, colq_hbm, rowq_hbm, cnts_hbm, out_hbm,
             colv, rowv, buf, cntv, x_sh, acc_sh):
    c = lax.axis_index("c")
    s = lax.axis_index("s")
    zeros16 = jnp.zeros((16,), jnp.float32)
    lanes = lax.iota(jnp.int32, 16)

    def xload(k, carry):
        r0 = s * ROWS_T + k * CH
        pltpu.sync_copy(x_hbm.at[pl.ds(r0, CH)], buf)
        pltpu.sync_copy(buf, x_sh.at[pl.ds(r0, CH)])
        return carry

    lax.fori_loop(0, ROWS_T // CH, xload, 0)

    def zero_buf():
        def zb(i, carry):
            for k in range(DIM // 16):
                buf[i, pl.ds(k * 16, 16)] = zeros16
            return carry

        lax.fori_loop(0, CH, zb, 0)

    def zero_acc():
        a0 = s * QROWS_T
        pltpu.sync_copy(buf, acc_sh.at[pl.ds(a0, CH)])
        pltpu.sync_copy(buf.at[pl.ds(0, QROWS_T - CH)],
                        acc_sh.at[pl.ds(a0 + CH, QROWS_T - CH)])

    zero_buf()
    zero_acc()
    plsc.subcore_barrier()

    for p in range(2):
        q = 2 * c + p
        for sl in range(2):
            t = 2 * s + sl
            pltpu.sync_copy(cnts_hbm.at[t], cntv)
            cnt = jnp.max(jnp.where(lanes == q, cntv[pl.ds(0, 16)], 0))
            trips = (cnt + CH - 1) // CH
            base_row = (q * NT + t) * CAPCH

            def chunk(ci, carry):
                pltpu.sync_copy(colq_hbm.at[base_row + ci], colv)
                pltpu.sync_copy(x_sh.at[colv], buf)
                pltpu.sync_copy(rowq_hbm.at[base_row + ci], rowv)
                pltpu.sync_copy(buf, acc_sh.at[rowv], add=True)
                return carry

            lax.fori_loop(0, trips, chunk, 0)
        plsc.subcore_barrier()

        a0 = s * QROWS_T
        o0 = q * QR + a0
        pltpu.sync_copy(acc_sh.at[pl.ds(a0, CH)], buf)
        pltpu.sync_copy(buf, out_hbm.at[pl.ds(o0, CH)])
        pltpu.sync_copy(acc_sh.at[pl.ds(a0 + CH, QROWS_T - CH)],
                        buf.at[pl.ds(0, QROWS_T - CH)])
        pltpu.sync_copy(buf.at[pl.ds(0, QROWS_T - CH)],
                        out_hbm.at[pl.ds(o0 + CH, QROWS_T - CH)])
        if p == 0:
            plsc.subcore_barrier()
            zero_buf()
            zero_acc()
            plsc.subcore_barrier()


_BR = 1280
_DEG_EPS = 1e-30


def _b1_body(x_ref, w_ref, dp_ref, o_ref):
    deg = jnp.maximum(dp_ref[0] + dp_ref[1], _DEG_EPS)
    y = lax.dot_general(
        x_ref[...], w_ref[...], (((1,), (1,)), ((), ())),
        preferred_element_type=jnp.float32, precision=lax.Precision.HIGHEST)
    o_ref[...] = y / deg


def _b2_body(s_ref, w_ref, dp_ref, o_ref):
    sacc = s_ref[...]
    nrm = jnp.maximum(
        jnp.sqrt(jnp.sum(sacc * sacc, axis=-1, keepdims=True)), 1e-12)
    u = jnp.maximum(sacc / nrm, 0.0)
    y = lax.dot_general(
        u, w_ref[...], (((1,), (1,)), ((), ())),
        preferred_element_type=jnp.float32, precision=lax.Precision.HIGHEST)
    o_ref[...] = y / jnp.maximum(dp_ref[0] + dp_ref[1], _DEG_EPS)


def _b3_body(s_ref, o_ref):
    sacc = s_ref[...]
    nrm = jnp.maximum(
        jnp.sqrt(jnp.sum(sacc * sacc, axis=-1, keepdims=True)), 1e-12)
    o_ref[...] = jnp.maximum(sacc / nrm, 0.0)


def _b1(x, w, degp):
    return pl.pallas_call(
        _b1_body,
        grid=(NPAD // _BR,),
        in_specs=[
            pl.BlockSpec((_BR, DIM), lambda i: (i, 0)),
            pl.BlockSpec((DIM, DIM), lambda i: (0, 0)),
            pl.BlockSpec((NC, _BR, 1), lambda i: (0, i, 0)),
        ],
        out_specs=pl.BlockSpec((_BR, DIM), lambda i: (i, 0)),
        out_shape=jax.ShapeDtypeStruct((NPAD, DIM), jnp.float32),
    )(x, w, degp)


def _b2(sp, w, degp):
    return pl.pallas_call(
        _b2_body,
        grid=(NPAD // _BR,),
        in_specs=[
            pl.BlockSpec((_BR, DIM), lambda i: (i, 0)),
            pl.BlockSpec((DIM, DIM), lambda i: (0, 0)),
            pl.BlockSpec((NC, _BR, 1), lambda i: (0, i, 0)),
        ],
        out_specs=pl.BlockSpec((_BR, DIM), lambda i: (i, 0)),
        out_shape=jax.ShapeDtypeStruct((NPAD, DIM), jnp.float32),
    )(sp, w, degp)


def _b3(sp):
    return pl.pallas_call(
        _b3_body,
        grid=(NPAD // _BR,),
        in_specs=[
            pl.BlockSpec((_BR, DIM), lambda i: (i, 0)),
        ],
        out_specs=pl.BlockSpec((_BR, DIM), lambda i: (i, 0)),
        out_shape=jax.ShapeDtypeStruct((NPAD, DIM), jnp.float32),
    )(sp)


def _debug_partition(row, col):
    t = jnp.arange(N_EDGES, dtype=jnp.int32) // E_TILE
    qid = row // QR
    g = qid * NT + t
    order = jnp.argsort(g, stable=True)
    gs = g[order]
    counts_g = jnp.bincount(g, length=NQ * NT).astype(jnp.int32)
    start = jnp.cumsum(counts_g) - counts_g
    pos = jnp.arange(N_EDGES, dtype=jnp.int32) - start[gs]
    dest = gs * CAP + pos
    colq = jnp.full((NQ * NT * CAP,), ZROW, jnp.int32).at[dest].set(col[order])
    rowq = jnp.zeros((NQ * NT * CAP,), jnp.int32).at[dest].set(
        (row - qid * QR)[order])
    cnts = jnp.zeros((NT, 16), jnp.int32)
    cnts = cnts.at[:, 0:NQ].set(counts_g.reshape(NQ, NT).T)
    return colq, rowq, cnts


def kernel(nodes_feature, edge_index, W0, W1):
    row = edge_index[0].astype(jnp.int32)
    col = edge_index[1].astype(jnp.int32)

    degp = _deg_fn(col).reshape(NC, NPAD, 1)
    colq, rowq, cnts = _debug_partition(row, col)
    colq2 = colq.reshape(-1, CH)
    rowq2 = rowq.reshape(-1, CH)

    x0 = jnp.pad(nodes_feature, ((0, NPAD - N_NODES), (0, 0)))
    x1 = _b1(x0, W0, degp)
    s1 = _spmm_fn(x1, colq2, rowq2, cnts)
    x2 = _b2(s1, W1, degp)
    s2 = _spmm_fn(x2, colq2, rowq2, cnts)
    out = _b3(s2)
    return out[:N_NODES]


# speedup vs baseline: 8.2195x; 8.2195x over previous
import functools

import jax
import jax.numpy as jnp
from jax import lax
from jax.experimental import pallas as pl
from jax.experimental.pallas import tpu as pltpu
from jax.experimental.pallas import tpu_sc as plsc

N_NODES = 10000
N_EDGES = 320000
DIM = 128

NC = 2
NS = 16
NT = NC * NS

NPAD = 10240
ZROW = NPAD - 1
NQ = 4
QR = NPAD // NQ
CH = 128
CAP = 10240
CAPCH = CAP // CH
E_TILE = N_EDGES // NT
HCH = 2000
ROWS_T = NPAD // NS
QROWS_T = QR // NS

_mesh = plsc.VectorSubcoreMesh(core_axis_name="c", subcore_axis_name="s")
_sc_params = pltpu.CompilerParams(needs_layout_passes=False)


@functools.partial(
    pl.kernel,
    out_type=jax.ShapeDtypeStruct((NC * NPAD,), jnp.float32),
    mesh=_mesh,
    compiler_params=_sc_params,
    scratch_types=[
        pltpu.VMEM((NPAD,), jnp.float32),
        pltpu.VMEM((HCH,), jnp.int32),
        pltpu.VMEM((NS, ROWS_T), jnp.float32),
        pltpu.VMEM((ROWS_T,), jnp.float32),
        pltpu.VMEM_SHARED((NS, NPAD), jnp.float32),
    ],
)
def _deg_fn(col_hbm, out_hbm, hist_v, colc_v, red_v, outb_v, stage_sh):
    c = lax.axis_index("c")
    s = lax.axis_index("s")
    gwid = c * NS + s
    zeros16 = jnp.zeros((16,), jnp.float32)
    ones16 = jnp.ones((16,), jnp.float32)

    def zbody(i, carry):
        hist_v[pl.ds(i * 16, 16)] = zeros16
        return carry

    lax.fori_loop(0, NPAD // 16, zbody, 0)

    def chunk_body(ci, carry):
        pltpu.sync_copy(col_hbm.at[pl.ds(gwid * E_TILE + ci * HCH, HCH)],
                        colc_v)

        def ibody(j, icarry):
            idx = colc_v[pl.ds(j * 16, 16)]
            plsc.addupdate_scatter(hist_v, [idx], ones16)
            return icarry

        lax.fori_loop(0, HCH // 16, ibody, 0)
        return carry

    lax.fori_loop(0, E_TILE // HCH, chunk_body, 0)

    pltpu.sync_copy(hist_v, stage_sh.at[s])
    plsc.subcore_barrier()

    pltpu.sync_copy(stage_sh.at[:, pl.ds(s * ROWS_T, ROWS_T)], red_v)

    def rbody(i, carry):
        acc = red_v[0, pl.ds(i * 16, 16)]
        for k in range(1, NS):
            acc = acc + red_v[k, pl.ds(i * 16, 16)]
        outb_v[pl.ds(i * 16, 16)] = acc
        return carry

    lax.fori_loop(0, ROWS_T // 16, rbody, 0)
    pltpu.sync_copy(outb_v, out_hbm.at[pl.ds(c * NPAD + s * ROWS_T, ROWS_T)])


@functools.partial(
    pl.kernel,
    out_type=[
        jax.ShapeDtypeStruct((NQ * NT * CAP,), jnp.int32),
        jax.ShapeDtypeStruct((NQ * NT * CAP,), jnp.int32),
        jax.ShapeDtypeStruct((NT, 16), jnp.int32),
    ],
    mesh=_mesh,
    compiler_params=_sc_params,
    scratch_types=[
        pltpu.VMEM((HCH,), jnp.int32),
        pltpu.VMEM((HCH,), jnp.int32),
        [pltpu.VMEM((CAP,), jnp.int32)] * NQ,
        [pltpu.VMEM((CAP,), jnp.int32)] * NQ,
        pltpu.VMEM((16,), jnp.int32),
    ],
)
def _part_fn(row_hbm, col_hbm, colq_hbm, rowq_hbm, cnts_hbm,
             cstage, rstage, qcol, qrow, cntv):
    c = lax.axis_index("c")
    s = lax.axis_index("s")
    gwid = c * NS + s
    colnull = jnp.full((16,), ZROW, jnp.int32)
    rownull = jnp.zeros((16,), jnp.int32)

    def chunk_body(ci, cnts):
        off = gwid * E_TILE + ci * HCH
        pltpu.sync_copy(col_hbm.at[pl.ds(off, HCH)], cstage)
        pltpu.sync_copy(row_hbm.at[pl.ds(off, HCH)], rstage)

        def ibody(j, icnts):
            cv = cstage[pl.ds(j * 16, 16)]
            rv = rstage[pl.ds(j * 16, 16)]
            out = []
            for q in range(NQ):
                lo = q * QR
                mq = (rv >= lo) & (rv < lo + QR)
                cq = icnts[q]
                pfx = plsc.cumsum(mq.astype(jnp.int32))
                pos = cq + pfx - 1
                plsc.store_scatter(qcol[q], [pos], cv, mask=mq)
                plsc.store_scatter(qrow[q], [pos], rv - lo, mask=mq)
                out.append(cq + jnp.max(pfx))
            return tuple(out)

        return lax.fori_loop(0, HCH // 16, ibody, cnts)

    z = jnp.int32(0)
    cnts = lax.fori_loop(0, E_TILE // HCH, chunk_body, (z, z, z, z))

    for q in range(NQ):
        for k in range(CH // 16):
            qcol[q][pl.ds(cnts[q] + k * 16, 16)] = colnull
            qrow[q][pl.ds(cnts[q] + k * 16, 16)] = rownull

    for q in range(NQ):
        base = (q * NT + gwid) * CAP
        pltpu.sync_copy(qcol[q], colq_hbm.at[pl.ds(base, CAP)])
        pltpu.sync_copy(qrow[q], rowq_hbm.at[pl.ds(base, CAP)])

    lanes = lax.iota(jnp.int32, 16)
    cvec = jnp.where(lanes == 0, cnts[0],
                     jnp.where(lanes == 1, cnts[1],
                               jnp.where(lanes == 2, cnts[2],
                                         jnp.where(lanes == 3, cnts[3], 0))))
    cntv[pl.ds(0, 16)] = cvec
    pltpu.sync_copy(cntv, cnts_hbm.at[gwid])


@functools.partial(
    pl.kernel,
    out_type=jax.ShapeDtypeStruct((NPAD, DIM), jnp.float32),
    mesh=_mesh,
    compiler_params=_sc_params,
    scratch_types=[
        pltpu.VMEM((CH,), jnp.int32),
        pltpu.VMEM((CH,), jnp.int32),
        pltpu.VMEM((CH, DIM), jnp.float32),
        pltpu.VMEM((16,), jnp.int32),
        pltpu.VMEM_SHARED((NPAD, DIM), jnp.float32),
        pltpu.VMEM_SHARED((QR, DIM), jnp.float32),
    ],
)
def _spmm_fn(x_hbm, colq_hbm, rowq_hbm, cnts_hbm, out_hbm,
             colv, rowv, buf, cntv, x_sh, acc_sh):
    c = lax.axis_index("c")
    s = lax.axis_index("s")
    zeros16 = jnp.zeros((16,), jnp.float32)
    lanes = lax.iota(jnp.int32, 16)

    def xload(k, carry):
        r0 = s * ROWS_T + k * CH
        pltpu.sync_copy(x_hbm.at[pl.ds(r0, CH)], buf)
        pltpu.sync_copy(buf, x_sh.at[pl.ds(r0, CH)])
        return carry

    lax.fori_loop(0, ROWS_T // CH, xload, 0)

    def zero_buf():
        def zb(i, carry):
            for k in range(DIM // 16):
                buf[i, pl.ds(k * 16, 16)] = zeros16
            return carry

        lax.fori_loop(0, CH, zb, 0)

    def zero_acc():
        a0 = s * QROWS_T
        pltpu.sync_copy(buf, acc_sh.at[pl.ds(a0, CH)])
        pltpu.sync_copy(buf.at[pl.ds(0, QROWS_T - CH)],
                        acc_sh.at[pl.ds(a0 + CH, QROWS_T - CH)])

    zero_buf()
    zero_acc()
    plsc.subcore_barrier()

    for p in range(2):
        q = 2 * c + p
        for sl in range(2):
            t = 2 * s + sl
            pltpu.sync_copy(cnts_hbm.at[t], cntv)
            cnt = jnp.max(jnp.where(lanes == q, cntv[pl.ds(0, 16)], 0))
            trips = (cnt + CH - 1) // CH
            base_row = (q * NT + t) * CAPCH

            def chunk(ci, carry):
                pltpu.sync_copy(colq_hbm.at[base_row + ci], colv)
                pltpu.sync_copy(x_sh.at[colv], buf)
                pltpu.sync_copy(rowq_hbm.at[base_row + ci], rowv)
                pltpu.sync_copy(buf, acc_sh.at[rowv], add=True)
                return carry

            lax.fori_loop(0, trips, chunk, 0)
        plsc.subcore_barrier()

        a0 = s * QROWS_T
        o0 = q * QR + a0
        pltpu.sync_copy(acc_sh.at[pl.ds(a0, CH)], buf)
        pltpu.sync_copy(buf, out_hbm.at[pl.ds(o0, CH)])
        pltpu.sync_copy(acc_sh.at[pl.ds(a0 + CH, QROWS_T - CH)],
                        buf.at[pl.ds(0, QROWS_T - CH)])
        pltpu.sync_copy(buf.at[pl.ds(0, QROWS_T - CH)],
                        out_hbm.at[pl.ds(o0 + CH, QROWS_T - CH)])
        if p == 0:
            plsc.subcore_barrier()
            zero_buf()
            zero_acc()
            plsc.subcore_barrier()


_BR = 1280
_DEG_EPS = 1e-30


def _b1_body(x_ref, w_ref, dp_ref, o_ref):
    deg = jnp.maximum(dp_ref[0] + dp_ref[1], _DEG_EPS)
    y = lax.dot_general(
        x_ref[...], w_ref[...], (((1,), (1,)), ((), ())),
        preferred_element_type=jnp.float32, precision=lax.Precision.HIGHEST)
    o_ref[...] = y / deg


def _b2_body(s_ref, w_ref, dp_ref, o_ref):
    sacc = s_ref[...]
    nrm = jnp.maximum(
        jnp.sqrt(jnp.sum(sacc * sacc, axis=-1, keepdims=True)), 1e-12)
    u = jnp.maximum(sacc / nrm, 0.0)
    y = lax.dot_general(
        u, w_ref[...], (((1,), (1,)), ((), ())),
        preferred_element_type=jnp.float32, precision=lax.Precision.HIGHEST)
    o_ref[...] = y / jnp.maximum(dp_ref[0] + dp_ref[1], _DEG_EPS)


def _b3_body(s_ref, o_ref):
    sacc = s_ref[...]
    nrm = jnp.maximum(
        jnp.sqrt(jnp.sum(sacc * sacc, axis=-1, keepdims=True)), 1e-12)
    o_ref[...] = jnp.maximum(sacc / nrm, 0.0)


def _b1(x, w, degp):
    return pl.pallas_call(
        _b1_body,
        grid=(NPAD // _BR,),
        in_specs=[
            pl.BlockSpec((_BR, DIM), lambda i: (i, 0)),
            pl.BlockSpec((DIM, DIM), lambda i: (0, 0)),
            pl.BlockSpec((NC, _BR, 1), lambda i: (0, i, 0)),
        ],
        out_specs=pl.BlockSpec((_BR, DIM), lambda i: (i, 0)),
        out_shape=jax.ShapeDtypeStruct((NPAD, DIM), jnp.float32),
    )(x, w, degp)


def _b2(sp, w, degp):
    return pl.pallas_call(
        _b2_body,
        grid=(NPAD // _BR,),
        in_specs=[
            pl.BlockSpec((_BR, DIM), lambda i: (i, 0)),
            pl.BlockSpec((DIM, DIM), lambda i: (0, 0)),
            pl.BlockSpec((NC, _BR, 1), lambda i: (0, i, 0)),
        ],
        out_specs=pl.BlockSpec((_BR, DIM), lambda i: (i, 0)),
        out_shape=jax.ShapeDtypeStruct((NPAD, DIM), jnp.float32),
    )(sp, w, degp)


def _b3(sp):
    return pl.pallas_call(
        _b3_body,
        grid=(NPAD // _BR,),
        in_specs=[
            pl.BlockSpec((_BR, DIM), lambda i: (i, 0)),
        ],
        out_specs=pl.BlockSpec((_BR, DIM), lambda i: (i, 0)),
        out_shape=jax.ShapeDtypeStruct((NPAD, DIM), jnp.float32),
    )(sp)


def _debug_partition(row, col):
    t = jnp.arange(N_EDGES, dtype=jnp.int32) // E_TILE
    qid = row // QR
    g = qid * NT + t
    order = jnp.argsort(g, stable=True)
    gs = g[order]
    counts_g = jnp.bincount(g, length=NQ * NT).astype(jnp.int32)
    start = jnp.cumsum(counts_g) - counts_g
    pos = jnp.arange(N_EDGES, dtype=jnp.int32) - start[gs]
    dest = gs * CAP + pos
    colq = jnp.full((NQ * NT * CAP,), ZROW, jnp.int32).at[dest].set(col[order])
    rowq = jnp.zeros((NQ * NT * CAP,), jnp.int32).at[dest].set(
        (row - qid * QR)[order])
    cnts = jnp.zeros((NT, 16), jnp.int32)
    cnts = cnts.at[:, 0:NQ].set(counts_g.reshape(NQ, NT).T)
    return colq, rowq, cnts


def kernel(nodes_feature, edge_index, W0, W1):
    row = edge_index[0].astype(jnp.int32)
    col = edge_index[1].astype(jnp.int32)

    degp = _deg_fn(col).reshape(NC, NPAD, 1)
    colq, rowq, cnts = _part_fn(row, col)
    colq2 = colq.reshape(-1, CH)
    rowq2 = rowq.reshape(-1, CH)

    x0 = jnp.pad(nodes_feature, ((0, NPAD - N_NODES), (0, 0)))
    x1 = _b1(x0, W0, degp)
    s1 = _spmm_fn(x1, colq2, rowq2, cnts)
    x2 = _b2(s1, W1, degp)
    s2 = _spmm_fn(x2, colq2, rowq2, cnts)
    out = _b3(s2)
    return out[:N_NODES]


# speedup vs baseline: 9.8044x; 1.1928x over previous
import functools

import jax
import jax.numpy as jnp
from jax import lax
from jax.experimental import pallas as pl
from jax.experimental.pallas import tpu as pltpu
from jax.experimental.pallas import tpu_sc as plsc

N_NODES = 10000
N_EDGES = 320000
DIM = 128

NC = 2
NS = 16
NT = NC * NS

NPAD = 10240
ZROW = NPAD - 1
NQ = 4
QR = NPAD // NQ
CH = 128
CAP = 10240
CAPCH = CAP // CH
E_TILE = N_EDGES // NT
HCH = 2000
ROWS_T = NPAD // NS
QROWS_T = QR // NS
SEG = 40

_mesh = plsc.VectorSubcoreMesh(core_axis_name="c", subcore_axis_name="s")
_sc_params = pltpu.CompilerParams(needs_layout_passes=False)


@functools.partial(
    pl.kernel,
    out_type=jax.ShapeDtypeStruct((NC * NPAD,), jnp.float32),
    mesh=_mesh,
    compiler_params=_sc_params,
    scratch_types=[
        pltpu.VMEM((NPAD,), jnp.float32),
        pltpu.VMEM((HCH,), jnp.int32),
        pltpu.VMEM((NS, ROWS_T), jnp.float32),
        pltpu.VMEM((ROWS_T,), jnp.float32),
        pltpu.VMEM_SHARED((NS, NPAD), jnp.float32),
    ],
)
def _deg_fn(col_hbm, out_hbm, hist_v, colc_v, red_v, outb_v, stage_sh):
    c = lax.axis_index("c")
    s = lax.axis_index("s")
    gwid = c * NS + s
    zeros16 = jnp.zeros((16,), jnp.float32)
    ones16 = jnp.ones((16,), jnp.float32)

    def zbody(i, carry):
        hist_v[pl.ds(i * 16, 16)] = zeros16
        return carry

    lax.fori_loop(0, NPAD // 16, zbody, 0)

    def chunk_body(ci, carry):
        pltpu.sync_copy(col_hbm.at[pl.ds(gwid * E_TILE + ci * HCH, HCH)],
                        colc_v)

        def ibody(j, icarry):
            idx = colc_v[pl.ds(j * 16, 16)]
            plsc.addupdate_scatter(hist_v, [idx], ones16)
            return icarry

        lax.fori_loop(0, HCH // 16, ibody, 0)
        return carry

    lax.fori_loop(0, E_TILE // HCH, chunk_body, 0)

    pltpu.sync_copy(hist_v, stage_sh.at[s])
    plsc.subcore_barrier()

    pltpu.sync_copy(stage_sh.at[:, pl.ds(s * ROWS_T, ROWS_T)], red_v)

    def rbody(i, carry):
        acc = red_v[0, pl.ds(i * 16, 16)]
        for k in range(1, NS):
            acc = acc + red_v[k, pl.ds(i * 16, 16)]
        outb_v[pl.ds(i * 16, 16)] = acc
        return carry

    lax.fori_loop(0, ROWS_T // 16, rbody, 0)
    pltpu.sync_copy(outb_v, out_hbm.at[pl.ds(c * NPAD + s * ROWS_T, ROWS_T)])


@functools.partial(
    pl.kernel,
    out_type=[
        jax.ShapeDtypeStruct((NQ * NT * CAP + SEG * CH,), jnp.int32),
        jax.ShapeDtypeStruct((NQ * NT * CAP + SEG * CH,), jnp.int32),
        jax.ShapeDtypeStruct((NT, 16), jnp.int32),
    ],
    mesh=_mesh,
    compiler_params=_sc_params,
    scratch_types=[
        pltpu.VMEM((HCH,), jnp.int32),
        pltpu.VMEM((HCH,), jnp.int32),
        [pltpu.VMEM((CAP,), jnp.int32)] * NQ,
        [pltpu.VMEM((CAP,), jnp.int32)] * NQ,
        pltpu.VMEM((16,), jnp.int32),
    ],
)
def _part_fn(row_hbm, col_hbm, colq_hbm, rowq_hbm, cnts_hbm,
             cstage, rstage, qcol, qrow, cntv):
    c = lax.axis_index("c")
    s = lax.axis_index("s")
    gwid = c * NS + s
    colnull = jnp.full((16,), ZROW, jnp.int32)
    rownull = jnp.zeros((16,), jnp.int32)

    def chunk_body(ci, cnts):
        off = gwid * E_TILE + ci * HCH
        pltpu.sync_copy(col_hbm.at[pl.ds(off, HCH)], cstage)
        pltpu.sync_copy(row_hbm.at[pl.ds(off, HCH)], rstage)

        def ibody(j, icnts):
            cv = cstage[pl.ds(j * 16, 16)]
            rv = rstage[pl.ds(j * 16, 16)]
            out = []
            for q in range(NQ):
                lo = q * QR
                mq = (rv >= lo) & (rv < lo + QR)
                cq = icnts[q]
                pfx = plsc.cumsum(mq.astype(jnp.int32))
                pos = cq + pfx - 1
                plsc.store_scatter(qcol[q], [pos], cv, mask=mq)
                plsc.store_scatter(qrow[q], [pos], rv - lo, mask=mq)
                out.append(cq + jnp.max(pfx))
            return tuple(out)

        return lax.fori_loop(0, HCH // 16, ibody, cnts)

    z = jnp.int32(0)
    cnts = lax.fori_loop(0, E_TILE // HCH, chunk_body, (z, z, z, z))

    for q in range(NQ):
        for k in range(CH // 16):
            qcol[q][pl.ds(cnts[q] + k * 16, 16)] = colnull
            qrow[q][pl.ds(cnts[q] + k * 16, 16)] = rownull

    for q in range(NQ):
        base = (q * NT + gwid) * CAP
        pltpu.sync_copy(qcol[q], colq_hbm.at[pl.ds(base, CAP)])
        pltpu.sync_copy(qrow[q], rowq_hbm.at[pl.ds(base, CAP)])

    lanes = lax.iota(jnp.int32, 16)
    cvec = jnp.where(lanes == 0, cnts[0],
                     jnp.where(lanes == 1, cnts[1],
                               jnp.where(lanes == 2, cnts[2],
                                         jnp.where(lanes == 3, cnts[3], 0))))
    cntv[pl.ds(0, 16)] = cvec
    pltpu.sync_copy(cntv, cnts_hbm.at[gwid])


@functools.partial(
    pl.kernel,
    out_type=jax.ShapeDtypeStruct((NPAD, DIM), jnp.float32),
    mesh=_mesh,
    compiler_params=_sc_params,
    scratch_types=[
        pltpu.VMEM((SEG, CH), jnp.int32),
        pltpu.VMEM((SEG, CH), jnp.int32),
        pltpu.VMEM((CH, DIM), jnp.float32),
        pltpu.VMEM((16,), jnp.int32),
        pltpu.VMEM_SHARED((NPAD, DIM), jnp.float32),
        pltpu.VMEM_SHARED((QR, DIM), jnp.float32),
    ],
)
def _spmm_fn(x_hbm, colq_hbm, rowq_hbm, cnts_hbm, out_hbm,
             colseg, rowseg, buf, cntv, x_sh, acc_sh):
    c = lax.axis_index("c")
    s = lax.axis_index("s")
    zeros16 = jnp.zeros((16,), jnp.float32)
    lanes = lax.iota(jnp.int32, 16)

    def xload(k, carry):
        r0 = s * ROWS_T + k * CH
        pltpu.sync_copy(x_hbm.at[pl.ds(r0, CH)], x_sh.at[pl.ds(r0, CH)])
        return carry

    lax.fori_loop(0, ROWS_T // CH, xload, 0)

    def zero_buf():
        def zb(i, carry):
            for k in range(DIM // 16):
                buf[i, pl.ds(k * 16, 16)] = zeros16
            return carry

        lax.fori_loop(0, CH, zb, 0)

    def zero_acc():
        a0 = s * QROWS_T
        pltpu.sync_copy(buf, acc_sh.at[pl.ds(a0, CH)])
        pltpu.sync_copy(buf.at[pl.ds(0, QROWS_T - CH)],
                        acc_sh.at[pl.ds(a0 + CH, QROWS_T - CH)])

    zero_buf()
    zero_acc()
    plsc.subcore_barrier()

    for p in range(2):
        q = 2 * c + p
        for sl in range(2):
            t = 2 * s + sl
            pltpu.sync_copy(cnts_hbm.at[t], cntv)
            cnt = jnp.max(jnp.where(lanes == q, cntv[pl.ds(0, 16)], 0))
            trips = (cnt + CH - 1) // CH
            base_row = (q * NT + t) * CAPCH

            nseg = (trips + SEG - 1) // SEG

            def seg_body(g, carry):
                srow = base_row + g * SEG
                pltpu.sync_copy(colq_hbm.at[pl.ds(srow, SEG)], colseg)
                pltpu.sync_copy(rowq_hbm.at[pl.ds(srow, SEG)], rowseg)
                in_seg = jnp.minimum(trips - g * SEG, SEG)

                def chunk(ci, c2):
                    pltpu.sync_copy(x_sh.at[colseg.at[ci]], buf)
                    pltpu.sync_copy(buf, acc_sh.at[rowseg.at[ci]], add=True)
                    return c2

                lax.fori_loop(0, in_seg, chunk, 0)
                return carry

            lax.fori_loop(0, nseg, seg_body, 0)
        plsc.subcore_barrier()

        a0 = s * QROWS_T
        o0 = q * QR + a0
        pltpu.sync_copy(acc_sh.at[pl.ds(a0, CH)], buf)
        pltpu.sync_copy(buf, out_hbm.at[pl.ds(o0, CH)])
        pltpu.sync_copy(acc_sh.at[pl.ds(a0 + CH, QROWS_T - CH)],
                        buf.at[pl.ds(0, QROWS_T - CH)])
        pltpu.sync_copy(buf.at[pl.ds(0, QROWS_T - CH)],
                        out_hbm.at[pl.ds(o0 + CH, QROWS_T - CH)])
        if p == 0:
            plsc.subcore_barrier()
            zero_buf()
            zero_acc()
            plsc.subcore_barrier()


_BR = 1280
_DEG_EPS = 1e-30


def _b1_body(x_ref, w_ref, dp_ref, o_ref):
    deg = jnp.maximum(dp_ref[0] + dp_ref[1], _DEG_EPS)
    y = lax.dot_general(
        x_ref[...], w_ref[...], (((1,), (1,)), ((), ())),
        preferred_element_type=jnp.float32, precision=lax.Precision.HIGHEST)
    o_ref[...] = y / deg


def _b2_body(s_ref, w_ref, dp_ref, o_ref):
    sacc = s_ref[...]
    nrm = jnp.maximum(
        jnp.sqrt(jnp.sum(sacc * sacc, axis=-1, keepdims=True)), 1e-12)
    u = jnp.maximum(sacc / nrm, 0.0)
    y = lax.dot_general(
        u, w_ref[...], (((1,), (1,)), ((), ())),
        preferred_element_type=jnp.float32, precision=lax.Precision.HIGHEST)
    o_ref[...] = y / jnp.maximum(dp_ref[0] + dp_ref[1], _DEG_EPS)


def _b3_body(s_ref, o_ref):
    sacc = s_ref[...]
    nrm = jnp.maximum(
        jnp.sqrt(jnp.sum(sacc * sacc, axis=-1, keepdims=True)), 1e-12)
    o_ref[...] = jnp.maximum(sacc / nrm, 0.0)


def _b1(x, w, degp):
    return pl.pallas_call(
        _b1_body,
        grid=(NPAD // _BR,),
        in_specs=[
            pl.BlockSpec((_BR, DIM), lambda i: (i, 0)),
            pl.BlockSpec((DIM, DIM), lambda i: (0, 0)),
            pl.BlockSpec((NC, _BR, 1), lambda i: (0, i, 0)),
        ],
        out_specs=pl.BlockSpec((_BR, DIM), lambda i: (i, 0)),
        out_shape=jax.ShapeDtypeStruct((NPAD, DIM), jnp.float32),
    )(x, w, degp)


def _b2(sp, w, degp):
    return pl.pallas_call(
        _b2_body,
        grid=(NPAD // _BR,),
        in_specs=[
            pl.BlockSpec((_BR, DIM), lambda i: (i, 0)),
            pl.BlockSpec((DIM, DIM), lambda i: (0, 0)),
            pl.BlockSpec((NC, _BR, 1), lambda i: (0, i, 0)),
        ],
        out_specs=pl.BlockSpec((_BR, DIM), lambda i: (i, 0)),
        out_shape=jax.ShapeDtypeStruct((NPAD, DIM), jnp.float32),
    )(sp, w, degp)


def _b3(sp):
    return pl.pallas_call(
        _b3_body,
        grid=(NPAD // _BR,),
        in_specs=[
            pl.BlockSpec((_BR, DIM), lambda i: (i, 0)),
        ],
        out_specs=pl.BlockSpec((_BR, DIM), lambda i: (i, 0)),
        out_shape=jax.ShapeDtypeStruct((NPAD, DIM), jnp.float32),
    )(sp)


def _debug_partition(row, col):
    t = jnp.arange(N_EDGES, dtype=jnp.int32) // E_TILE
    qid = row // QR
    g = qid * NT + t
    order = jnp.argsort(g, stable=True)
    gs = g[order]
    counts_g = jnp.bincount(g, length=NQ * NT).astype(jnp.int32)
    start = jnp.cumsum(counts_g) - counts_g
    pos = jnp.arange(N_EDGES, dtype=jnp.int32) - start[gs]
    dest = gs * CAP + pos
    colq = jnp.full((NQ * NT * CAP,), ZROW, jnp.int32).at[dest].set(col[order])
    rowq = jnp.zeros((NQ * NT * CAP,), jnp.int32).at[dest].set(
        (row - qid * QR)[order])
    cnts = jnp.zeros((NT, 16), jnp.int32)
    cnts = cnts.at[:, 0:NQ].set(counts_g.reshape(NQ, NT).T)
    return colq, rowq, cnts


def kernel(nodes_feature, edge_index, W0, W1):
    row = edge_index[0].astype(jnp.int32)
    col = edge_index[1].astype(jnp.int32)

    degp = _deg_fn(col).reshape(NC, NPAD, 1)
    colq, rowq, cnts = _part_fn(row, col)
    colq2 = colq.reshape(-1, CH)
    rowq2 = rowq.reshape(-1, CH)

    x0 = jnp.pad(nodes_feature, ((0, NPAD - N_NODES), (0, 0)))
    x1 = _b1(x0, W0, degp)
    s1 = _spmm_fn(x1, colq2, rowq2, cnts)
    x2 = _b2(s1, W1, degp)
    s2 = _spmm_fn(x2, colq2, rowq2, cnts)
    out = _b3(s2)
    return out[:N_NODES]
